# Initial kernel scaffold; baseline (speedup 1.0000x reference)
import jax
import jax.numpy as jnp
from jax.experimental import pallas as pl


def kernel(batchInput, sourceEmbedding_weight):
    raise NotImplementedError("write your pallas kernel here")


# speedup vs baseline: 1.2855x; 1.2855x over previous
import functools

import jax
import jax.numpy as jnp
from jax import lax
from jax.experimental import pallas as pl
from jax.experimental.pallas import tpu as pltpu
from jax.experimental.pallas import tpu_sc as plsc

BATCH = 16384
HIST = 50
DIM = 32
ROW_W = 128
N_ROWS = BATCH * HIST // ROW_W

_info = plsc.get_sparse_core_info()
NC, NS = _info.num_cores, _info.num_subcores
NW = NC * NS
ROWS_PER_W = N_ROWS // NW
K = 8
ITERS = ROWS_PER_W // K


def _make_kernel():
  mesh = plsc.VectorSubcoreMesh(core_axis_name="c", subcore_axis_name="s")

  @functools.partial(
      pl.kernel,
      mesh=mesh,
      compiler_params=pltpu.CompilerParams(use_tc_tiling_on_sc=False),
      out_type=jax.ShapeDtypeStruct((N_ROWS, ROW_W, DIM), jnp.float32),
      scratch_types=[
          pltpu.VMEM((K, ROW_W), jnp.int32),
          pltpu.VMEM((K, ROW_W, DIM), jnp.float32),
          pltpu.SemaphoreType.DMA,
      ],
  )
  def gather_kernel(idx_hbm, table_hbm, out_hbm, idx_v, rows_v, sem):
    wid = lax.axis_index("s") * NC + lax.axis_index("c")
    row_base = wid * ROWS_PER_W

    def step(i, carry):
      rb = row_base + i * K
      pltpu.sync_copy(idx_hbm.at[pl.ds(rb, K)], idx_v)
      copies = [
          pltpu.async_copy(table_hbm.at[idx_v.at[j]], rows_v.at[j], sem)
          for j in range(K)
      ]
      for c in copies:
        c.wait()
      pltpu.sync_copy(rows_v, out_hbm.at[pl.ds(rb, K)])
      return carry

    lax.fori_loop(0, ITERS, step, 0)

  return gather_kernel


_kernel = _make_kernel()


@jax.jit
def kernel(batchInput, sourceEmbedding_weight):
  idx = batchInput.reshape(N_ROWS, ROW_W).astype(jnp.int32)
  out = _kernel(idx, sourceEmbedding_weight)
  return out.reshape(BATCH, HIST, DIM)


# speedup vs baseline: 1.3522x; 1.0519x over previous
import functools

import jax
import jax.numpy as jnp
from jax import lax
from jax.experimental import pallas as pl
from jax.experimental.pallas import tpu as pltpu
from jax.experimental.pallas import tpu_sc as plsc

BATCH = 16384
HIST = 50
DIM = 32
BBLK = 128
N_UNITS = HIST * (BATCH // BBLK)
IDX_ROWS = N_UNITS

_info = plsc.get_sparse_core_info()
NC, NS = _info.num_cores, _info.num_subcores
NW = NC * NS
UNITS_PER_W = N_UNITS // NW
Q = 4
QUADS = UNITS_PER_W // Q


def _make_kernel():
  mesh = plsc.VectorSubcoreMesh(core_axis_name="c", subcore_axis_name="s")

  @functools.partial(
      pl.kernel,
      mesh=mesh,
      compiler_params=pltpu.CompilerParams(
          use_tc_tiling_on_sc=False, needs_layout_passes=False
      ),
      out_type=jax.ShapeDtypeStruct(
          (HIST, DIM // 8, BATCH // BBLK, 8, BBLK), jnp.float32
      ),
      scratch_types=[
          pltpu.VMEM((Q, BBLK), jnp.int32),
          pltpu.VMEM((BBLK, DIM), jnp.float32),
          pltpu.VMEM((BBLK, DIM), jnp.float32),
          pltpu.VMEM((BBLK, DIM), jnp.float32),
          pltpu.VMEM((BBLK, DIM), jnp.float32),
          pltpu.VMEM((DIM, BBLK), jnp.float32),
          pltpu.SemaphoreType.DMA,
      ],
  )
  def gather_kernel(idx_hbm, table_hbm, out_hbm, idx_v, r0, r1, r2, r3,
                    cols_v, gsem):
    wid = lax.axis_index("s") * NC + lax.axis_index("c")
    base = wid * UNITS_PER_W
    lane = lax.broadcasted_iota(jnp.int32, (16,), 0)

    def transpose_unit(rows, u):
      for c in range(DIM):
        cvec = jnp.full((16,), c, dtype=jnp.int32)
        for g in range(8):
          rvec = lane + (g * 16)
          v = plsc.load_gather(rows, [rvec, cvec])
          cols_v[c, pl.ds(g * 16, 16)] = v

      h = u // BBLK
      b_hi = u % BBLK
      for c_hi in range(DIM // 8):
        pltpu.sync_copy(
            cols_v.at[pl.ds(c_hi * 8, 8)], out_hbm.at[h, c_hi, b_hi]
        )

    rbufs = [r0, r1, r2, r3]

    def quad(t, carry):
      q0 = base + t * Q
      pltpu.sync_copy(idx_hbm.at[pl.ds(q0, Q)], idx_v)
      copies = [
          pltpu.async_copy(table_hbm.at[idx_v.at[j]], rbufs[j], gsem)
          for j in range(Q)
      ]
      for j in range(Q):
        copies[j].wait()
        transpose_unit(rbufs[j], q0 + j)
      return carry

    lax.fori_loop(0, QUADS, quad, 0)

  return gather_kernel


_kernel = _make_kernel()


@jax.jit
def kernel(batchInput, sourceEmbedding_weight):
  idx_t = batchInput.astype(jnp.int32).T.reshape(IDX_ROWS, BBLK)
  out5d = _kernel(idx_t, sourceEmbedding_weight)
  return out5d.transpose(2, 4, 0, 1, 3).reshape(BATCH, HIST, DIM)


# speedup vs baseline: 1.6474x; 1.2184x over previous
import functools

import jax
import jax.numpy as jnp
from jax import lax
from jax.experimental import pallas as pl
from jax.experimental.pallas import tpu as pltpu
from jax.experimental.pallas import tpu_sc as plsc

BATCH = 16384
HIST = 50
DIM = 32
BBLK = 128
N_UNITS = HIST * (BATCH // BBLK)
IDX_ROWS = N_UNITS

_info = plsc.get_sparse_core_info()
NC, NS = _info.num_cores, _info.num_subcores
NW = NC * NS
UNITS_PER_W = N_UNITS // NW
Q = 4
QUADS = UNITS_PER_W // Q


def _make_kernel():
  mesh = plsc.VectorSubcoreMesh(core_axis_name="c", subcore_axis_name="s")

  scratch = [pltpu.VMEM((UNITS_PER_W, BBLK), jnp.int32)]
  scratch += [pltpu.VMEM((BBLK, DIM), jnp.float32) for _ in range(2 * Q)]
  scratch += [
      pltpu.VMEM((DIM // 8, 8, BBLK), jnp.float32) for _ in range(2 * Q)
  ]
  scratch += [pltpu.SemaphoreType.DMA for _ in range(4)]

  @functools.partial(
      pl.kernel,
      mesh=mesh,
      compiler_params=pltpu.CompilerParams(
          use_tc_tiling_on_sc=False, needs_layout_passes=False
      ),
      out_type=jax.ShapeDtypeStruct(
          (HIST, DIM // 8, BATCH // BBLK, 8, BBLK), jnp.float32
      ),
      scratch_types=scratch,
  )
  def gather_kernel(idx_hbm, table_hbm, out_hbm, idx_v, *bufs):
    rbufs = bufs[0:2 * Q]
    cbufs = bufs[2 * Q:4 * Q]
    gsems = bufs[4 * Q:4 * Q + 2]
    osems = bufs[4 * Q + 2:4 * Q + 4]

    wid = lax.axis_index("s") * NC + lax.axis_index("c")
    base = wid * UNITS_PER_W
    lane = lax.broadcasted_iota(jnp.int32, (16,), 0)

    pltpu.sync_copy(idx_hbm.at[pl.ds(base, UNITS_PER_W)], idx_v)

    def fire(t, s):
      for j in range(Q):
        pltpu.async_copy(
            table_hbm.at[idx_v.at[t * Q + j]], rbufs[s * Q + j], gsems[s]
        )

    def gwait(s, j):
      pltpu.make_async_copy(
          table_hbm.at[pl.ds(0, BBLK)], rbufs[s * Q + j], gsems[s]
      ).wait()

    def owait(s, j):
      pltpu.make_async_copy(
          table_hbm.at[pl.ds(0, BBLK)], cbufs[s * Q + j], osems[s]
      ).wait()

    def consume(t, s, drain_old):
      for j in range(Q):
        gwait(s, j)
        if drain_old:
          owait(s, j)
        rows, cols = rbufs[s * Q + j], cbufs[s * Q + j]

        def col_body(c, carry, rows=rows, cols=cols):
          cvec = jnp.full((16,), 1, dtype=jnp.int32) * c
          for g in range(8):
            v = plsc.load_gather(rows, [lane + g * 16, cvec])
            cols[c // 8, c % 8, pl.ds(g * 16, 16)] = v
          return carry

        lax.fori_loop(0, DIM, col_body, 0)
        u = base + t * Q + j
        h = u // BBLK
        b_hi = u % BBLK
        pltpu.async_copy(cols, out_hbm.at[h, :, b_hi], osems[s])

    fire(0, 0)
    fire(1, 1)
    consume(0, 0, False)
    fire(2, 0)
    consume(1, 1, False)

    def pair(i, carry):
      t0 = 2 * i
      fire(t0 + 1, 1)
      consume(t0, 0, True)
      fire(t0 + 2, 0)
      consume(t0 + 1, 1, True)
      return carry

    lax.fori_loop(1, QUADS // 2 - 1, pair, 0)

    fire(QUADS - 1, 1)
    consume(QUADS - 2, 0, True)
    consume(QUADS - 1, 1, True)
    for j in range(Q):
      owait(0, j)
      owait(1, j)

  return gather_kernel


_kernel = _make_kernel()


@jax.jit
def kernel(batchInput, sourceEmbedding_weight):
  idx_t = batchInput.astype(jnp.int32).T.reshape(IDX_ROWS, BBLK)
  out5d = _kernel(idx_t, sourceEmbedding_weight)
  return out5d.transpose(2, 4, 0, 1, 3).reshape(BATCH, HIST, DIM)


# speedup vs baseline: 1.6483x; 1.0005x over previous
import functools

import jax
import jax.numpy as jnp
from jax import lax
from jax.experimental import pallas as pl
from jax.experimental.pallas import tpu as pltpu
from jax.experimental.pallas import tpu_sc as plsc

BATCH = 16384
HIST = 50
DIM = 32
BBLK = 128
N_UNITS = HIST * (BATCH // BBLK)

_info = plsc.get_sparse_core_info()
NC, NS = _info.num_cores, _info.num_subcores
NW = NC * NS
UNITS_PER_W = N_UNITS // NW
Q = 4
QROW = Q * BBLK
QUADS = UNITS_PER_W // Q
IDX_ROWS = N_UNITS // Q


def _make_kernel():
  mesh = plsc.VectorSubcoreMesh(core_axis_name="c", subcore_axis_name="s")

  scratch = [pltpu.VMEM((QUADS, QROW), jnp.int32)]
  scratch += [pltpu.VMEM((QROW, DIM), jnp.float32) for _ in range(2)]
  scratch += [
      pltpu.VMEM((DIM // 8, 8, BBLK), jnp.float32) for _ in range(2 * Q)
  ]
  scratch += [pltpu.SemaphoreType.DMA for _ in range(4)]

  @functools.partial(
      pl.kernel,
      mesh=mesh,
      compiler_params=pltpu.CompilerParams(
          use_tc_tiling_on_sc=False, needs_layout_passes=False
      ),
      out_type=jax.ShapeDtypeStruct(
          (HIST, DIM // 8, BATCH // BBLK, 8, BBLK), jnp.float32
      ),
      scratch_types=scratch,
  )
  def gather_kernel(idx_hbm, table_hbm, out_hbm, idx_v, *bufs):
    rbufs = bufs[0:2]
    cbufs = bufs[2:2 + 2 * Q]
    gsems = bufs[2 + 2 * Q:4 + 2 * Q]
    osems = bufs[4 + 2 * Q:6 + 2 * Q]

    wid = lax.axis_index("s") * NC + lax.axis_index("c")
    base = wid * QUADS
    lane = lax.broadcasted_iota(jnp.int32, (16,), 0)

    pltpu.sync_copy(idx_hbm.at[pl.ds(base, QUADS)], idx_v)

    def fire(t, s):
      pltpu.async_copy(table_hbm.at[idx_v.at[t]], rbufs[s], gsems[s])

    def gwait(s):
      pltpu.make_async_copy(
          table_hbm.at[pl.ds(0, QROW)], rbufs[s], gsems[s]
      ).wait()

    def owait(s, j):
      pltpu.make_async_copy(
          table_hbm.at[pl.ds(0, BBLK)], cbufs[s * Q + j], osems[s]
      ).wait()

    def consume(t, s, drain_old):
      gwait(s)
      rows = rbufs[s]
      for j in range(Q):
        if drain_old:
          owait(s, j)
        cols = cbufs[s * Q + j]

        def col_body(c, carry, rows=rows, cols=cols, j=j):
          cvec = jnp.full((16,), 1, dtype=jnp.int32) * c
          for g in range(8):
            v = plsc.load_gather(rows, [lane + (j * BBLK + g * 16), cvec])
            cols[c // 8, c % 8, pl.ds(g * 16, 16)] = v
          return carry

        lax.fori_loop(0, DIM, col_body, 0)

        u = (base + t) * Q + j
        h = u // BBLK
        b_hi = u % BBLK
        pltpu.async_copy(cols, out_hbm.at[h, :, b_hi], osems[s])

    fire(0, 0)
    fire(1, 1)
    consume(0, 0, False)
    fire(2, 0)
    consume(1, 1, False)

    def pair(i, carry):
      t0 = 2 * i
      fire(t0 + 1, 1)
      consume(t0, 0, True)
      fire(t0 + 2, 0)
      consume(t0 + 1, 1, True)
      return carry

    lax.fori_loop(1, QUADS // 2 - 1, pair, 0)

    fire(QUADS - 1, 1)
    consume(QUADS - 2, 0, True)
    consume(QUADS - 1, 1, True)
    for j in range(Q):
      owait(0, j)
      owait(1, j)

  return gather_kernel


_kernel = _make_kernel()


@jax.jit
def kernel(batchInput, sourceEmbedding_weight):
  idx_t = batchInput.astype(jnp.int32).T.reshape(IDX_ROWS, QROW)
  out5d = _kernel(idx_t, sourceEmbedding_weight)
  return out5d.transpose(2, 4, 0, 1, 3).reshape(BATCH, HIST, DIM)
